# baseline (speedup 1.0000x reference)
import jax
import jax.numpy as jnp
from jax import lax
from jax.experimental import pallas as pl
from jax.experimental.pallas import tpu as pltpu
from jax.experimental.pallas import tpu_sc as plsc

_K = 5
_LABEL_SMOOTHING = 0.5
_SMOOTH_LOSS_WEIGHT = 0.5
_BR = 8
_LOG2E = 1.4426950408889634
_NSTREAMS = 4
_UNROLL = 8

_V = 100000
_V_TC = 67200
_SC_CORES = 2
_SC_SUBCORES = 16
_NW = _SC_CORES * _SC_SUBCORES
_SC_CHUNK = 8192


def _row_kernel(lbl_ref, x_ref, v5_ref, c5_ref, ssum_ref, amax_ref, lblv_ref):
    br = x_ref.shape[0]
    v = x_ref.shape[1]
    neg = jnp.float32(-jnp.inf)

    nfull = v // 128
    tail_w = v - nfull * 128

    def fresh_state():
        return (
            jnp.full((br, 128), neg, jnp.float32),
            jnp.full((br, 128), neg, jnp.float32),
            jnp.full((br, 128), neg, jnp.float32),
            jnp.full((br, 128), neg, jnp.float32),
            jnp.full((br, 128), neg, jnp.float32),
            jnp.full((br, 128), jnp.int32(nfull), jnp.int32),
            jnp.zeros((br, 128), jnp.float32),
        )

    states = [fresh_state() for _ in range(_NSTREAMS)]

    if tail_w:
        xt = x_ref[:, nfull * 128 :]
        padf = jnp.full((br, 128 - tail_w), neg, jnp.float32)
        t1 = jnp.concatenate([xt, padf], axis=1)
        s = jnp.concatenate(
            [jnp.exp2(xt * _LOG2E), jnp.zeros((br, 128 - tail_w), jnp.float32)],
            axis=1,
        )
        st = states[0]
        states[0] = (t1, st[1], st[2], st[3], st[4], st[5], s)

    def insert(state, j):
        t1, t2, t3, t4, t5, i1, s = state
        xj = x_ref[:, pl.ds(pl.multiple_of(j * 128, 128), 128)]
        upd = xj >= t1
        i1 = jnp.where(upd, jnp.int32(j) if isinstance(j, int) else j, i1)
        d = jnp.minimum(t1, xj)
        t1 = jnp.maximum(t1, xj)
        d2 = jnp.minimum(t2, d)
        t2 = jnp.maximum(t2, d)
        d3 = jnp.minimum(t3, d2)
        t3 = jnp.maximum(t3, d2)
        d4 = jnp.minimum(t4, d3)
        t4 = jnp.maximum(t4, d3)
        t5 = jnp.maximum(t5, d4)
        s = s + jnp.exp2(xj * _LOG2E)
        return t1, t2, t3, t4, t5, i1, s

    iters = nfull // _UNROLL
    rem = nfull - iters * _UNROLL

    for j in range(nfull - 1, nfull - rem - 1, -1):
        states[j % _NSTREAMS] = insert(states[j % _NSTREAMS], j)

    def body(it, carry):
        states = [tuple(c) for c in carry]
        base = (iters - 1 - it) * _UNROLL
        for u in range(_UNROLL - 1, -1, -1):
            sid = u % _NSTREAMS
            states[sid] = insert(states[sid], base + u)
        return tuple(states)

    if iters:
        states = list(
            jax.lax.fori_loop(0, iters, body, tuple(tuple(s) for s in states))
        )

    t1s = [st[0] for st in states]
    t1m = t1s[0]
    for t in t1s[1:]:
        t1m = jnp.maximum(t1m, t)
    m = jnp.max(t1m, axis=1, keepdims=True)

    ssum_l = states[0][6]
    for st in states[1:]:
        ssum_l = ssum_l + st[6]
    ssum_ref[...] = jnp.sum(ssum_l, axis=1, keepdims=True)

    lanes = jax.lax.broadcasted_iota(jnp.int32, (br, 128), 1)
    big = jnp.int32(2**30)
    amax = jnp.full((br, 1), big, jnp.int32)
    for st in states:
        gidx = st[5] * 128 + lanes
        cand_idx = jnp.where(st[0] == m, gidx, big)
        amax = jnp.minimum(amax, jnp.min(cand_idx, axis=1, keepdims=True))
    amax_ref[...] = amax

    cand = jnp.concatenate(
        [st[i] for st in states for i in range(5)], axis=1
    )
    t = m
    for k in range(_K):
        c = jnp.sum(jnp.where(cand == t, 1.0, 0.0), axis=1, keepdims=True)
        v5_ref[:, k : k + 1] = t
        c5_ref[:, k : k + 1] = c
        t = jnp.max(jnp.where(cand < t, cand, neg), axis=1, keepdims=True)

    lane128 = jax.lax.broadcasted_iota(jnp.int32, (1, 128), 1)
    if tail_w:
        lane_t = jax.lax.broadcasted_iota(jnp.int32, (1, tail_w), 1)
    lvals = []
    for r in range(br):
        idx = lbl_ref[r, 0]
        jl = jnp.minimum(idx // 128, jnp.int32(nfull - 1))
        v0 = x_ref[pl.ds(r, 1), pl.ds(pl.multiple_of(jl * 128, 128), 128)]
        off = idx - jl * 128
        val = jnp.sum(jnp.where(lane128 == off, v0, 0.0), axis=1, keepdims=True)
        if tail_w:
            off_t = idx - jnp.int32(nfull * 128)
            val = val + jnp.sum(
                jnp.where(lane_t == off_t, xt[r : r + 1, :], 0.0),
                axis=1,
                keepdims=True,
            )
        lvals.append(val)
    lblv_ref[...] = jnp.concatenate(lvals, axis=0)


def _sc_worker(x1_ref, lbl_hbm, out_hbm, lblout_hbm, buf, lblbuf, resbuf, lblres, gbuf, sem):
    wid = lax.axis_index("s") * _SC_CORES + lax.axis_index("c")
    rows_per_w = 1024 // _NW
    base = wid * rows_per_w
    lanes = lax.iota(jnp.int32, 16)
    neg = jnp.float32(-jnp.inf)
    v_sc = _V - _V_TC

    chunks = []
    off = 0
    while off < v_sc:
        c = min(_SC_CHUNK, v_sc - off)
        chunks.append((off, c))
        off += c

    def row_body(r, carry_dummy):
        row = base + r
        rowstart = row * _V + _V_TC
        st = (
            jnp.full((16,), neg, jnp.float32),
            jnp.full((16,), neg, jnp.float32),
            jnp.full((16,), neg, jnp.float32),
            jnp.full((16,), neg, jnp.float32),
            jnp.full((16,), neg, jnp.float32),
            jnp.zeros((16,), jnp.float32),
            jnp.zeros((16,), jnp.float32),
        )
        for coff, csz in chunks:
            pltpu.sync_copy(
                x1_ref.at[pl.ds(rowstart + coff, csz)], buf.at[pl.ds(0, csz)]
            )
            gbase = jnp.int32(coff // 16)

            def inner(g, st):
                t1, t2, t3, t4, t5, i1, s = st
                xv = buf[pl.ds(g * 16, 16)]
                i1 = jnp.where(xv > t1, (g + gbase).astype(jnp.float32), i1)
                d = jnp.minimum(t1, xv)
                t1 = jnp.maximum(t1, xv)
                d2 = jnp.minimum(t2, d)
                t2 = jnp.maximum(t2, d)
                d3 = jnp.minimum(t3, d2)
                t3 = jnp.maximum(t3, d2)
                d4 = jnp.minimum(t4, d3)
                t4 = jnp.maximum(t4, d3)
                t5 = jnp.maximum(t5, d4)
                s = s + jnp.exp(xv)
                return (t1, t2, t3, t4, t5, i1, s)

            st = lax.fori_loop(0, csz // 16, inner, st)

        t1, t2, t3, t4, t5, i1, s = st
        m = jnp.max(t1)
        gidxf = i1 * 16.0 + lanes.astype(jnp.float32) + jnp.float32(_V_TC)
        amaxf = -jnp.max(jnp.where(t1 == m, -gidxf, jnp.float32(-1e9)))
        ssum = jnp.sum(s)

        ts = [t1, t2, t3, t4, t5]
        t = m
        p = jnp.zeros((16,), jnp.float32)
        for k in range(_K):
            c = jnp.float32(0.0)
            nt = neg
            for tj in ts:
                c = c + jnp.sum(jnp.where(tj == t, 1.0, 0.0))
                nt = jnp.maximum(nt, jnp.max(jnp.where(tj < t, tj, neg)))
            p = jnp.where(lanes == k, t, p)
            p = jnp.where(lanes == _K + k, c, p)
            t = nt
        p = jnp.where(lanes == 10, ssum, p)
        p = jnp.where(lanes == 11, amaxf, p)
        resbuf[pl.ds(r * 16, 16)] = p
        return carry_dummy

    lax.fori_loop(0, rows_per_w, row_body, jnp.int32(0))
    pltpu.sync_copy(resbuf, out_hbm.at[pl.ds(base * 16, rows_per_w * 16)])

    pltpu.sync_copy(lbl_hbm.at[pl.ds(base, rows_per_w)], lblbuf)
    for k in range(rows_per_w // 16):
        lv = lblbuf[pl.ds(k * 16, 16)]
        rows = (base + k * 16 + lanes) * jnp.int32(_V) + lv
        pltpu.async_copy(x1_ref.at[rows], gbuf, sem).wait()
        vals = jnp.where(lv >= jnp.int32(_V_TC), gbuf[...], jnp.float32(0.0))
        lblres[pl.ds(k * 16, 16)] = vals
    pltpu.sync_copy(lblres, lblout_hbm.at[pl.ds(base, rows_per_w)])


def _sc_call(x1d, labels):
    b = labels.shape[0]
    rows_per_w = b // _NW
    mesh = plsc.VectorSubcoreMesh(
        core_axis_name="c",
        subcore_axis_name="s",
        num_cores=_SC_CORES,
        num_subcores=_SC_SUBCORES,
    )
    return pl.kernel(
        _sc_worker,
        out_type=[
            jax.ShapeDtypeStruct((b * 16,), jnp.float32),
            jax.ShapeDtypeStruct((b,), jnp.float32),
        ],
        mesh=mesh,
        compiler_params=pltpu.CompilerParams(needs_layout_passes=False),
        scratch_types=[
            pltpu.VMEM((_SC_CHUNK,), jnp.float32),
            pltpu.VMEM((rows_per_w,), jnp.int32),
            pltpu.VMEM((rows_per_w * 16,), jnp.float32),
            pltpu.VMEM((rows_per_w,), jnp.float32),
            pltpu.VMEM((16,), jnp.float32),
            pltpu.SemaphoreType.DMA,
        ],
    )(x1d, labels)


def _combine_kernel(
    lbl_ref, v5t_ref, c5t_ref, ssumt_ref, amaxt_ref, lblvt_ref, scp_ref, sclbl_ref, out_ref
):
    neg = jnp.float32(-jnp.inf)
    v5t = v5t_ref[...]
    c5t = c5t_ref[...]
    scp = scp_ref[...]
    v_sc = scp[:, 0:5]
    c_sc = scp[:, 5:10]
    ssum = ssumt_ref[...] + scp[:, 10:11]
    amax_sc = scp[:, 11:12].astype(jnp.int32)
    lse = jnp.log(ssum)
    m_tc = v5t[:, 0:1]
    m_sc = scp[:, 0:1]
    amax = jnp.where(m_tc >= m_sc, amaxt_ref[...], amax_sc)
    lblv = lblvt_ref[...] + sclbl_ref[...]

    vall = jnp.concatenate([v5t, v_sc], axis=1)
    call = jnp.concatenate([c5t, c_sc], axis=1)
    t = jnp.maximum(m_tc, m_sc)
    rem = jnp.full_like(t, jnp.float32(_K))
    acc = jnp.zeros_like(t)
    for _ in range(_K):
        cnt = jnp.sum(jnp.where(vall == t, call, 0.0), axis=1, keepdims=True)
        take = jnp.minimum(cnt, rem)
        acc = acc + jnp.where(take > 0.0, t * take, 0.0)
        rem = rem - take
        t = jnp.max(jnp.where(vall < t, vall, neg), axis=1, keepdims=True)

    hard = lse - lblv
    uniform = (lse - acc / _K) * _LABEL_SMOOTHING
    smooth = uniform + (1.0 - _LABEL_SMOOTHING) * hard
    corr = (amax == lbl_ref[...]).astype(jnp.float32)

    n = jnp.float32(corr.shape[0])
    nc = jnp.sum(corr)
    ni = n - nc
    sw = _SMOOTH_LOSS_WEIGHT * (nc / n)
    hw = (1.0 - _SMOOTH_LOSS_WEIGHT) * (ni / n)
    tot = sw + hw
    sw = sw / tot
    hw = hw / tot
    hard_loss = jnp.sum(corr * hard) * hw / jnp.maximum(nc, 1.0)
    smooth_loss = jnp.sum((1.0 - corr) * smooth) * sw / jnp.maximum(ni, 1.0)
    out_ref[...] = jnp.reshape(hard_loss + smooth_loss, (1, 1))


def kernel(logits, labels):
    b, v = logits.shape
    lbl2 = labels.reshape(b, 1)
    x1d = logits.reshape(-1)
    nb = b // _BR

    scp, sclbl = _sc_call(x1d, labels)
    scp = scp.reshape(b, 16)
    sclbl = sclbl.reshape(b, 1)

    v5, c5, ssum, amax, lblv = pl.pallas_call(
        _row_kernel,
        grid=(nb,),
        in_specs=[
            pl.BlockSpec((_BR, 1), lambda i: (i, 0), memory_space=pltpu.SMEM),
            pl.BlockSpec((_BR, _V_TC), lambda i: (i, 0)),
        ],
        out_specs=[
            pl.BlockSpec((_BR, _K), lambda i: (i, 0)),
            pl.BlockSpec((_BR, _K), lambda i: (i, 0)),
            pl.BlockSpec((_BR, 1), lambda i: (i, 0)),
            pl.BlockSpec((_BR, 1), lambda i: (i, 0)),
            pl.BlockSpec((_BR, 1), lambda i: (i, 0)),
        ],
        out_shape=[
            jax.ShapeDtypeStruct((b, _K), jnp.float32),
            jax.ShapeDtypeStruct((b, _K), jnp.float32),
            jax.ShapeDtypeStruct((b, 1), jnp.float32),
            jax.ShapeDtypeStruct((b, 1), jnp.int32),
            jax.ShapeDtypeStruct((b, 1), jnp.float32),
        ],
    )(lbl2, logits)

    out = pl.pallas_call(
        _combine_kernel,
        out_shape=jax.ShapeDtypeStruct((1, 1), jnp.float32),
    )(lbl2, v5, c5, ssum, amax, lblv, scp, sclbl)
    return out[0, 0]


# speedup vs baseline: 1.0100x; 1.0100x over previous
import jax
import jax.numpy as jnp
from jax import lax
from jax.experimental import pallas as pl
from jax.experimental.pallas import tpu as pltpu
from jax.experimental.pallas import tpu_sc as plsc

_K = 5
_LABEL_SMOOTHING = 0.5
_SMOOTH_LOSS_WEIGHT = 0.5
_BR = 8
_LOG2E = 1.4426950408889634
_NSTREAMS = 4
_UNROLL = 8

_V = 100000
_V_TC = 91776
_SC_CORES = 2
_SC_SUBCORES = 16
_NW = _SC_CORES * _SC_SUBCORES
_SC_CHUNK = 8192


def _row_kernel(lbl_ref, x_ref, v5_ref, c5_ref, ssum_ref, amax_ref, lblv_ref):
    br = x_ref.shape[0]
    v = x_ref.shape[1]
    neg = jnp.float32(-jnp.inf)

    nfull = v // 128
    tail_w = v - nfull * 128

    def fresh_state():
        return (
            jnp.full((br, 128), neg, jnp.float32),
            jnp.full((br, 128), neg, jnp.float32),
            jnp.full((br, 128), neg, jnp.float32),
            jnp.full((br, 128), neg, jnp.float32),
            jnp.full((br, 128), neg, jnp.float32),
            jnp.full((br, 128), jnp.int32(nfull), jnp.int32),
            jnp.zeros((br, 128), jnp.float32),
        )

    states = [fresh_state() for _ in range(_NSTREAMS)]

    if tail_w:
        xt = x_ref[:, nfull * 128 :]
        padf = jnp.full((br, 128 - tail_w), neg, jnp.float32)
        t1 = jnp.concatenate([xt, padf], axis=1)
        s = jnp.concatenate(
            [jnp.exp2(xt * _LOG2E), jnp.zeros((br, 128 - tail_w), jnp.float32)],
            axis=1,
        )
        st = states[0]
        states[0] = (t1, st[1], st[2], st[3], st[4], st[5], s)

    def insert(state, j):
        t1, t2, t3, t4, t5, i1, s = state
        xj = x_ref[:, pl.ds(pl.multiple_of(j * 128, 128), 128)]
        upd = xj >= t1
        i1 = jnp.where(upd, jnp.int32(j) if isinstance(j, int) else j, i1)
        d = jnp.minimum(t1, xj)
        t1 = jnp.maximum(t1, xj)
        d2 = jnp.minimum(t2, d)
        t2 = jnp.maximum(t2, d)
        d3 = jnp.minimum(t3, d2)
        t3 = jnp.maximum(t3, d2)
        d4 = jnp.minimum(t4, d3)
        t4 = jnp.maximum(t4, d3)
        t5 = jnp.maximum(t5, d4)
        s = s + jnp.exp2(xj * _LOG2E)
        return t1, t2, t3, t4, t5, i1, s

    iters = nfull // _UNROLL
    rem = nfull - iters * _UNROLL

    for j in range(nfull - 1, nfull - rem - 1, -1):
        states[j % _NSTREAMS] = insert(states[j % _NSTREAMS], j)

    def body(it, carry):
        states = [tuple(c) for c in carry]
        base = (iters - 1 - it) * _UNROLL
        for u in range(_UNROLL - 1, -1, -1):
            sid = u % _NSTREAMS
            states[sid] = insert(states[sid], base + u)
        return tuple(states)

    if iters:
        states = list(
            jax.lax.fori_loop(0, iters, body, tuple(tuple(s) for s in states))
        )

    t1s = [st[0] for st in states]
    t1m = t1s[0]
    for t in t1s[1:]:
        t1m = jnp.maximum(t1m, t)
    m = jnp.max(t1m, axis=1, keepdims=True)

    ssum_l = states[0][6]
    for st in states[1:]:
        ssum_l = ssum_l + st[6]
    ssum_ref[...] = jnp.sum(ssum_l, axis=1, keepdims=True)

    lanes = jax.lax.broadcasted_iota(jnp.int32, (br, 128), 1)
    big = jnp.int32(2**30)
    amax = jnp.full((br, 1), big, jnp.int32)
    for st in states:
        gidx = st[5] * 128 + lanes
        cand_idx = jnp.where(st[0] == m, gidx, big)
        amax = jnp.minimum(amax, jnp.min(cand_idx, axis=1, keepdims=True))
    amax_ref[...] = amax

    cand = jnp.concatenate(
        [st[i] for st in states for i in range(5)], axis=1
    )
    t = m
    for k in range(_K):
        c = jnp.sum(jnp.where(cand == t, 1.0, 0.0), axis=1, keepdims=True)
        v5_ref[:, k : k + 1] = t
        c5_ref[:, k : k + 1] = c
        t = jnp.max(jnp.where(cand < t, cand, neg), axis=1, keepdims=True)

    lane128 = jax.lax.broadcasted_iota(jnp.int32, (1, 128), 1)
    if tail_w:
        lane_t = jax.lax.broadcasted_iota(jnp.int32, (1, tail_w), 1)
    lvals = []
    for r in range(br):
        idx = lbl_ref[r, 0]
        jl = jnp.minimum(idx // 128, jnp.int32(nfull - 1))
        v0 = x_ref[pl.ds(r, 1), pl.ds(pl.multiple_of(jl * 128, 128), 128)]
        off = idx - jl * 128
        val = jnp.sum(jnp.where(lane128 == off, v0, 0.0), axis=1, keepdims=True)
        if tail_w:
            off_t = idx - jnp.int32(nfull * 128)
            val = val + jnp.sum(
                jnp.where(lane_t == off_t, xt[r : r + 1, :], 0.0),
                axis=1,
                keepdims=True,
            )
        lvals.append(val)
    lblv_ref[...] = jnp.concatenate(lvals, axis=0)


def _sc_worker(x1_ref, lbl_hbm, out_hbm, lblout_hbm, buf, lblbuf, resbuf, lblres, gbuf, sem):
    wid = lax.axis_index("s") * _SC_CORES + lax.axis_index("c")
    rows_per_w = 1024 // _NW
    base = wid * rows_per_w
    lanes = lax.iota(jnp.int32, 16)
    neg = jnp.float32(-jnp.inf)
    v_sc = _V - _V_TC

    chunks = []
    off = 0
    while off < v_sc:
        c = min(_SC_CHUNK, v_sc - off)
        chunks.append((off, c))
        off += c

    def row_body(r, carry_dummy):
        row = base + r
        rowstart = row * _V + _V_TC
        st = (
            jnp.full((16,), neg, jnp.float32),
            jnp.full((16,), neg, jnp.float32),
            jnp.full((16,), neg, jnp.float32),
            jnp.full((16,), neg, jnp.float32),
            jnp.full((16,), neg, jnp.float32),
            jnp.zeros((16,), jnp.float32),
            jnp.zeros((16,), jnp.float32),
        )
        for coff, csz in chunks:
            pltpu.sync_copy(
                x1_ref.at[pl.ds(rowstart + coff, csz)], buf.at[pl.ds(0, csz)]
            )
            gbase = jnp.int32(coff // 16)

            def inner(g, st):
                t1, t2, t3, t4, t5, i1, s = st
                xv = buf[pl.ds(g * 16, 16)]
                i1 = jnp.where(xv > t1, (g + gbase).astype(jnp.float32), i1)
                d = jnp.minimum(t1, xv)
                t1 = jnp.maximum(t1, xv)
                d2 = jnp.minimum(t2, d)
                t2 = jnp.maximum(t2, d)
                d3 = jnp.minimum(t3, d2)
                t3 = jnp.maximum(t3, d2)
                d4 = jnp.minimum(t4, d3)
                t4 = jnp.maximum(t4, d3)
                t5 = jnp.maximum(t5, d4)
                s = s + jnp.exp(xv)
                return (t1, t2, t3, t4, t5, i1, s)

            st = lax.fori_loop(0, csz // 16, inner, st)

        t1, t2, t3, t4, t5, i1, s = st
        m = jnp.max(t1)
        gidxf = i1 * 16.0 + lanes.astype(jnp.float32) + jnp.float32(_V_TC)
        amaxf = -jnp.max(jnp.where(t1 == m, -gidxf, jnp.float32(-1e9)))
        ssum = jnp.sum(s)

        ts = [t1, t2, t3, t4, t5]
        t = m
        p = jnp.zeros((16,), jnp.float32)
        for k in range(_K):
            c = jnp.float32(0.0)
            nt = neg
            for tj in ts:
                c = c + jnp.sum(jnp.where(tj == t, 1.0, 0.0))
                nt = jnp.maximum(nt, jnp.max(jnp.where(tj < t, tj, neg)))
            p = jnp.where(lanes == k, t, p)
            p = jnp.where(lanes == _K + k, c, p)
            t = nt
        p = jnp.where(lanes == 10, ssum, p)
        p = jnp.where(lanes == 11, amaxf, p)
        resbuf[pl.ds(r * 16, 16)] = p
        return carry_dummy

    lax.fori_loop(0, rows_per_w, row_body, jnp.int32(0))
    pltpu.sync_copy(resbuf, out_hbm.at[pl.ds(base * 16, rows_per_w * 16)])

    pltpu.sync_copy(lbl_hbm.at[pl.ds(base, rows_per_w)], lblbuf)
    for k in range(rows_per_w // 16):
        lv = lblbuf[pl.ds(k * 16, 16)]
        rows = (base + k * 16 + lanes) * jnp.int32(_V) + lv
        pltpu.async_copy(x1_ref.at[rows], gbuf, sem).wait()
        vals = jnp.where(lv >= jnp.int32(_V_TC), gbuf[...], jnp.float32(0.0))
        lblres[pl.ds(k * 16, 16)] = vals
    pltpu.sync_copy(lblres, lblout_hbm.at[pl.ds(base, rows_per_w)])


def _sc_call(x1d, labels):
    b = labels.shape[0]
    rows_per_w = b // _NW
    mesh = plsc.VectorSubcoreMesh(
        core_axis_name="c",
        subcore_axis_name="s",
        num_cores=_SC_CORES,
        num_subcores=_SC_SUBCORES,
    )
    return pl.kernel(
        _sc_worker,
        out_type=[
            jax.ShapeDtypeStruct((b * 16,), jnp.float32),
            jax.ShapeDtypeStruct((b,), jnp.float32),
        ],
        mesh=mesh,
        compiler_params=pltpu.CompilerParams(needs_layout_passes=False),
        scratch_types=[
            pltpu.VMEM((_SC_CHUNK,), jnp.float32),
            pltpu.VMEM((rows_per_w,), jnp.int32),
            pltpu.VMEM((rows_per_w * 16,), jnp.float32),
            pltpu.VMEM((rows_per_w,), jnp.float32),
            pltpu.VMEM((16,), jnp.float32),
            pltpu.SemaphoreType.DMA,
        ],
    )(x1d, labels)


def _combine_kernel(
    lbl_ref, v5t_ref, c5t_ref, ssumt_ref, amaxt_ref, lblvt_ref, scp_ref, sclbl_ref, out_ref
):
    neg = jnp.float32(-jnp.inf)
    v5t = v5t_ref[...]
    c5t = c5t_ref[...]
    scp = scp_ref[...]
    v_sc = scp[:, 0:5]
    c_sc = scp[:, 5:10]
    ssum = ssumt_ref[...] + scp[:, 10:11]
    amax_sc = scp[:, 11:12].astype(jnp.int32)
    lse = jnp.log(ssum)
    m_tc = v5t[:, 0:1]
    m_sc = scp[:, 0:1]
    amax = jnp.where(m_tc >= m_sc, amaxt_ref[...], amax_sc)
    lblv = lblvt_ref[...] + sclbl_ref[...]

    vall = jnp.concatenate([v5t, v_sc], axis=1)
    call = jnp.concatenate([c5t, c_sc], axis=1)
    t = jnp.maximum(m_tc, m_sc)
    rem = jnp.full_like(t, jnp.float32(_K))
    acc = jnp.zeros_like(t)
    for _ in range(_K):
        cnt = jnp.sum(jnp.where(vall == t, call, 0.0), axis=1, keepdims=True)
        take = jnp.minimum(cnt, rem)
        acc = acc + jnp.where(take > 0.0, t * take, 0.0)
        rem = rem - take
        t = jnp.max(jnp.where(vall < t, vall, neg), axis=1, keepdims=True)

    hard = lse - lblv
    uniform = (lse - acc / _K) * _LABEL_SMOOTHING
    smooth = uniform + (1.0 - _LABEL_SMOOTHING) * hard
    corr = (amax == lbl_ref[...]).astype(jnp.float32)

    n = jnp.float32(corr.shape[0])
    nc = jnp.sum(corr)
    ni = n - nc
    sw = _SMOOTH_LOSS_WEIGHT * (nc / n)
    hw = (1.0 - _SMOOTH_LOSS_WEIGHT) * (ni / n)
    tot = sw + hw
    sw = sw / tot
    hw = hw / tot
    hard_loss = jnp.sum(corr * hard) * hw / jnp.maximum(nc, 1.0)
    smooth_loss = jnp.sum((1.0 - corr) * smooth) * sw / jnp.maximum(ni, 1.0)
    out_ref[...] = jnp.reshape(hard_loss + smooth_loss, (1, 1))


def kernel(logits, labels):
    b, v = logits.shape
    lbl2 = labels.reshape(b, 1)
    x1d = logits.reshape(-1)
    nb = b // _BR

    scp, sclbl = _sc_call(x1d, labels)
    scp = scp.reshape(b, 16)
    sclbl = sclbl.reshape(b, 1)

    v5, c5, ssum, amax, lblv = pl.pallas_call(
        _row_kernel,
        grid=(nb,),
        in_specs=[
            pl.BlockSpec((_BR, 1), lambda i: (i, 0), memory_space=pltpu.SMEM),
            pl.BlockSpec((_BR, _V_TC), lambda i: (i, 0)),
        ],
        out_specs=[
            pl.BlockSpec((_BR, _K), lambda i: (i, 0)),
            pl.BlockSpec((_BR, _K), lambda i: (i, 0)),
            pl.BlockSpec((_BR, 1), lambda i: (i, 0)),
            pl.BlockSpec((_BR, 1), lambda i: (i, 0)),
            pl.BlockSpec((_BR, 1), lambda i: (i, 0)),
        ],
        out_shape=[
            jax.ShapeDtypeStruct((b, _K), jnp.float32),
            jax.ShapeDtypeStruct((b, _K), jnp.float32),
            jax.ShapeDtypeStruct((b, 1), jnp.float32),
            jax.ShapeDtypeStruct((b, 1), jnp.int32),
            jax.ShapeDtypeStruct((b, 1), jnp.float32),
        ],
    )(lbl2, logits)

    out = pl.pallas_call(
        _combine_kernel,
        out_shape=jax.ShapeDtypeStruct((1, 1), jnp.float32),
    )(lbl2, v5, c5, ssum, amax, lblv, scp, sclbl)
    return out[0, 0]
